# baseline (speedup 1.0000x reference)
import jax
import jax.numpy as jnp
from jax.experimental import pallas as pl
from jax.experimental.pallas import tpu as pltpu

N = 4096
F = 256
L = 5532
Q = 5000
SCALAR = 30.0
LN2 = 0.6931471805599453
C = SCALAR / LN2
BC = 1024
NLB = (L + BC - 1) // BC
NQB = (Q + BC - 1) // BC
TL = L - (NLB - 1) * BC
TQ = Q - (NQB - 1) * BC
PAD = (BC - TL) + (BC - TQ)
NEG = -1e30


def _oim_body(x_ref, lut_ref, cq_ref, lbl_ref, out_ref, xc_s, m_s, s_s, g_s):
    j = pl.program_id(0)

    @pl.when(j == 0)
    def _init():
        xc_s[...] = x_ref[...] * C
        m_s[...] = jnp.full((1, N), NEG, dtype=jnp.float32)
        s_s[...] = jnp.zeros((1, N), dtype=jnp.float32)
        g_s[...] = jnp.zeros((1, N), dtype=jnp.float32)

    @pl.when(j == NLB - 1)
    def _zero_lut_tail():
        lut_ref[TL:, :] = jnp.zeros((BC - TL, F), jnp.float32)

    @pl.when(j == NQB - 1)
    def _zero_cq_tail():
        cq_ref[TQ:, :] = jnp.zeros((BC - TQ, F), jnp.float32)

    def accum(lgc):
        m_old = m_s[...]
        m_new = jnp.maximum(m_old, jnp.max(lgc, axis=0, keepdims=True))
        s_s[...] = (s_s[...] * jnp.exp2(m_old - m_new)
                    + jnp.sum(jnp.exp2(lgc - m_new), axis=0, keepdims=True))
        m_s[...] = m_new

    lg_l = jax.lax.dot_general(
        lut_ref[...], xc_s[...], (((1,), (1,)), ((), ())),
        preferred_element_type=jnp.float32)

    col = j * BC + jax.lax.broadcasted_iota(jnp.int32, (BC, 1), 0)
    hit = col == lbl_ref[...].astype(jnp.int32)
    g_s[...] += jnp.sum(jnp.where(hit, lg_l, 0.0), axis=0, keepdims=True)
    accum(lg_l)

    @pl.when(j < NQB)
    def _cq_part():
        lg_q = jax.lax.dot_general(
            cq_ref[...], xc_s[...], (((1,), (1,)), ((), ())),
            preferred_element_type=jnp.float32)
        accum(lg_q)

    @pl.when(j == NLB - 1)
    def _finish():
        m = m_s[...]
        s = s_s[...] - PAD * jnp.exp2(-m)
        valid = lbl_ref[...] >= 0.0
        nll = LN2 * (m - g_s[...]) + jnp.log(s)
        loss_sum = jnp.sum(jnp.where(valid, nll, 0.0), keepdims=True)
        cnt = jnp.sum(valid.astype(jnp.float32), keepdims=True)
        out_ref[...] = loss_sum / jnp.maximum(cnt, 1.0)


@jax.jit
def _oim_loss(inputs, label_f, lut, cq):
    out = pl.pallas_call(
        _oim_body,
        grid=(NLB,),
        in_specs=[
            pl.BlockSpec((N, F), lambda j: (0, 0)),
            pl.BlockSpec((BC, F), lambda j: (j, 0)),
            pl.BlockSpec((BC, F), lambda j: (jnp.minimum(j, NQB - 1), 0)),
            pl.BlockSpec((1, N), lambda j: (0, 0)),
        ],
        out_specs=pl.BlockSpec((1, 1), lambda j: (0, 0)),
        out_shape=jax.ShapeDtypeStruct((1, 1), jnp.float32),
        scratch_shapes=[
            pltpu.VMEM((N, F), jnp.float32),
            pltpu.VMEM((1, N), jnp.float32),
            pltpu.VMEM((1, N), jnp.float32),
            pltpu.VMEM((1, N), jnp.float32),
        ],
        compiler_params=pltpu.CompilerParams(
            dimension_semantics=("arbitrary",)),
    )(inputs, lut, cq, label_f)
    return out[0, 0]


def kernel(inputs, roi_label, detectionscore, lut, cq):
    label_f = (roi_label.reshape(1, -1) - 1).astype(jnp.float32)
    loss = _oim_loss(inputs, label_f, lut, cq)
    return (loss, lut)


# speedup vs baseline: 1.0224x; 1.0224x over previous
import jax
import jax.numpy as jnp
from jax.experimental import pallas as pl
from jax.experimental.pallas import tpu as pltpu

N = 4096
F = 256
L = 5532
Q = 5000
SCALAR = 30.0
LN2 = 0.6931471805599453
C = SCALAR / LN2
BC = 512
NLB = (L + BC - 1) // BC
NQB = (Q + BC - 1) // BC
TL = L - (NLB - 1) * BC
TQ = Q - (NQB - 1) * BC
PAD = (BC - TL) + (BC - TQ)
NEG = -1e30


def _oim_body(x_ref, lut_ref, cq_ref, lbl_ref, out_ref, xc_s, m_s, s_s, g_s):
    j = pl.program_id(0)

    @pl.when(j == 0)
    def _init():
        xc_s[...] = x_ref[...] * C
        m_s[...] = jnp.full((1, N), NEG, dtype=jnp.float32)
        s_s[...] = jnp.zeros((1, N), dtype=jnp.float32)
        g_s[...] = jnp.zeros((1, N), dtype=jnp.float32)

    @pl.when(j == NLB - 1)
    def _zero_lut_tail():
        lut_ref[TL:, :] = jnp.zeros((BC - TL, F), jnp.float32)

    @pl.when(j == NQB - 1)
    def _zero_cq_tail():
        cq_ref[TQ:, :] = jnp.zeros((BC - TQ, F), jnp.float32)

    def accum(lgc):
        m_old = m_s[...]
        m_new = jnp.maximum(m_old, jnp.max(lgc, axis=0, keepdims=True))
        s_s[...] = (s_s[...] * jnp.exp2(m_old - m_new)
                    + jnp.sum(jnp.exp2(lgc - m_new), axis=0, keepdims=True))
        m_s[...] = m_new

    lg_l = jax.lax.dot_general(
        lut_ref[...], xc_s[...], (((1,), (1,)), ((), ())),
        preferred_element_type=jnp.float32)

    col = j * BC + jax.lax.broadcasted_iota(jnp.int32, (BC, 1), 0)
    hit = col == lbl_ref[...].astype(jnp.int32)
    g_s[...] += jnp.sum(jnp.where(hit, lg_l, 0.0), axis=0, keepdims=True)
    accum(lg_l)

    @pl.when(j < NQB)
    def _cq_part():
        lg_q = jax.lax.dot_general(
            cq_ref[...], xc_s[...], (((1,), (1,)), ((), ())),
            preferred_element_type=jnp.float32)
        accum(lg_q)

    @pl.when(j == NLB - 1)
    def _finish():
        m = m_s[...]
        s = s_s[...] - PAD * jnp.exp2(-m)
        valid = lbl_ref[...] >= 0.0
        nll = LN2 * (m - g_s[...]) + jnp.log(s)
        loss_sum = jnp.sum(jnp.where(valid, nll, 0.0), keepdims=True)
        cnt = jnp.sum(valid.astype(jnp.float32), keepdims=True)
        out_ref[...] = loss_sum / jnp.maximum(cnt, 1.0)


@jax.jit
def _oim_loss(inputs, label_f, lut, cq):
    out = pl.pallas_call(
        _oim_body,
        grid=(NLB,),
        in_specs=[
            pl.BlockSpec((N, F), lambda j: (0, 0)),
            pl.BlockSpec((BC, F), lambda j: (j, 0)),
            pl.BlockSpec((BC, F), lambda j: (jnp.minimum(j, NQB - 1), 0)),
            pl.BlockSpec((1, N), lambda j: (0, 0)),
        ],
        out_specs=pl.BlockSpec((1, 1), lambda j: (0, 0)),
        out_shape=jax.ShapeDtypeStruct((1, 1), jnp.float32),
        scratch_shapes=[
            pltpu.VMEM((N, F), jnp.float32),
            pltpu.VMEM((1, N), jnp.float32),
            pltpu.VMEM((1, N), jnp.float32),
            pltpu.VMEM((1, N), jnp.float32),
        ],
        compiler_params=pltpu.CompilerParams(
            dimension_semantics=("arbitrary",)),
    )(inputs, lut, cq, label_f)
    return out[0, 0]


def kernel(inputs, roi_label, detectionscore, lut, cq):
    label_f = (roi_label.reshape(1, -1) - 1).astype(jnp.float32)
    loss = _oim_loss(inputs, label_f, lut, cq)
    return (loss, lut)


# speedup vs baseline: 1.1272x; 1.1025x over previous
import jax
import jax.numpy as jnp
from jax.experimental import pallas as pl
from jax.experimental.pallas import tpu as pltpu

N = 4096
F = 256
L = 5532
Q = 5000
SCALAR = 30.0
LN2 = 0.6931471805599453
C = SCALAR / LN2
BC = 512
NLB = (L + BC - 1) // BC
NQB = (Q + BC - 1) // BC
TL = L - (NLB - 1) * BC
TQ = Q - (NQB - 1) * BC
PAD = (BC - TL) + (BC - TQ) + BC
NEG = -1e30


def _oim_body(x_ref, lut_ref, cq_ref, lbl_ref, out_ref, xc_s, m_s, s_s, g_s):
    j = pl.program_id(0)

    @pl.when(j == 0)
    def _init():
        xc_s[...] = x_ref[...] * C
        m_s[...] = jnp.full((1, N), NEG, dtype=jnp.float32)
        s_s[...] = jnp.zeros((1, N), dtype=jnp.float32)
        g_s[...] = jnp.zeros((1, N), dtype=jnp.float32)

    @pl.when(j == NLB - 1)
    def _zero_lut_tail():
        lut_ref[TL:, :] = jnp.zeros((BC - TL, F), jnp.float32)
        cq_ref[...] = jnp.zeros((BC, F), jnp.float32)

    @pl.when(j == NQB - 1)
    def _zero_cq_tail():
        cq_ref[TQ:, :] = jnp.zeros((BC - TQ, F), jnp.float32)

    lg_l = jax.lax.dot_general(
        lut_ref[...], xc_s[...], (((1,), (1,)), ((), ())),
        preferred_element_type=jnp.float32)
    lg_q = jax.lax.dot_general(
        cq_ref[...], xc_s[...], (((1,), (1,)), ((), ())),
        preferred_element_type=jnp.float32)

    col = j * BC + jax.lax.broadcasted_iota(jnp.int32, (BC, 1), 0)
    hit = col == lbl_ref[...].astype(jnp.int32)
    g_s[...] += jnp.sum(jnp.where(hit, lg_l, 0.0), axis=0, keepdims=True)

    m_old = m_s[...]
    bm = jnp.maximum(jnp.max(lg_l, axis=0, keepdims=True),
                     jnp.max(lg_q, axis=0, keepdims=True))
    m_new = jnp.maximum(m_old, bm)
    s_s[...] = (s_s[...] * jnp.exp2(m_old - m_new)
                + jnp.sum(jnp.exp2(lg_l - m_new), axis=0, keepdims=True)
                + jnp.sum(jnp.exp2(lg_q - m_new), axis=0, keepdims=True))
    m_s[...] = m_new

    @pl.when(j == NLB - 1)
    def _finish():
        m = m_s[...]
        s = s_s[...] - PAD * jnp.exp2(-m)
        valid = lbl_ref[...] >= 0.0
        nll = LN2 * (m - g_s[...]) + jnp.log(s)
        loss_sum = jnp.sum(jnp.where(valid, nll, 0.0), keepdims=True)
        cnt = jnp.sum(valid.astype(jnp.float32), keepdims=True)
        out_ref[...] = loss_sum / jnp.maximum(cnt, 1.0)


@jax.jit
def _oim_loss(inputs, label_f, lut, cq):
    out = pl.pallas_call(
        _oim_body,
        grid=(NLB,),
        in_specs=[
            pl.BlockSpec((N, F), lambda j: (0, 0)),
            pl.BlockSpec((BC, F), lambda j: (j, 0)),
            pl.BlockSpec((BC, F), lambda j: (jnp.minimum(j, NQB - 1), 0)),
            pl.BlockSpec((1, N), lambda j: (0, 0)),
        ],
        out_specs=pl.BlockSpec((1, 1), lambda j: (0, 0)),
        out_shape=jax.ShapeDtypeStruct((1, 1), jnp.float32),
        scratch_shapes=[
            pltpu.VMEM((N, F), jnp.float32),
            pltpu.VMEM((1, N), jnp.float32),
            pltpu.VMEM((1, N), jnp.float32),
            pltpu.VMEM((1, N), jnp.float32),
        ],
        compiler_params=pltpu.CompilerParams(
            dimension_semantics=("arbitrary",)),
    )(inputs, lut, cq, label_f)
    return out[0, 0]


def kernel(inputs, roi_label, detectionscore, lut, cq):
    label_f = (roi_label.reshape(1, -1) - 1).astype(jnp.float32)
    loss = _oim_loss(inputs, label_f, lut, cq)
    return (loss, lut)
